# Initial kernel scaffold; baseline (speedup 1.0000x reference)
import functools

import jax
import jax.numpy as jnp
from jax import lax
from jax.experimental import pallas as pl
from jax.experimental.pallas import tpu as pltpu
from jax.experimental.pallas import tpu_sc as plsc

N = 10000
E = 320000
D = 128

NC = 2
NS = 16
NW = NC * NS
GW = 128
EPW = E // NW
SCH = 80
SNCH = EPW // SCH
RPT = N // NS
ZR = 125

_PREC = lax.Precision.HIGHEST


def _dot(a, b):
    return lax.dot_general(a, b, (((1,), (0,)), ((), ())), precision=_PREC,
                           preferred_element_type=jnp.float32)


def _project_body(nf_ref, w0s_ref, w0r_ref, ps_ref, pr_ref):
    nf = nf_ref[...]
    ps_ref[...] = _dot(nf, w0s_ref[...])
    pr_ref[...] = _dot(nf, w0r_ref[...])


def _project(nf, w0s, w0r, bn=2000):
    grid = (N // bn,)
    return pl.pallas_call(
        _project_body,
        grid=grid,
        in_specs=[
            pl.BlockSpec((bn, D), lambda i: (i, 0)),
            pl.BlockSpec((D, D), lambda i: (0, 0)),
            pl.BlockSpec((D, D), lambda i: (0, 0)),
        ],
        out_specs=[
            pl.BlockSpec((bn, D), lambda i: (i, 0)),
            pl.BlockSpec((bn, D), lambda i: (i, 0)),
        ],
        out_shape=[
            jax.ShapeDtypeStruct((N, D), jnp.float32),
            jax.ShapeDtypeStruct((N, D), jnp.float32),
        ],
    )(nf, w0s, w0r)


def _sc_gather(ps, pr, senders, receivers):
    mesh = plsc.VectorSubcoreMesh(core_axis_name="core",
                                  subcore_axis_name="subcore")

    @functools.partial(
        pl.kernel,
        out_type=(
            jax.ShapeDtypeStruct((E, D), jnp.float32),
            jax.ShapeDtypeStruct((E, D), jnp.float32),
        ),
        mesh=mesh,
    )
    def gk(ps_hbm, pr_hbm, s_hbm, r_hbm, gs_hbm, gr_hbm):
        def body(si_v, ri_v, gs_v, gr_v):
            pltpu.sync_copy(ps_hbm.at[si_v.at[0]], gs_v)
            pltpu.sync_copy(pr_hbm.at[ri_v.at[0]], gr_v)

        pltpu.emit_pipeline(
            body,
            grid=(E // GW,),
            in_specs=[
                pl.BlockSpec((1, GW), lambda i: (0, i)),
                pl.BlockSpec((1, GW), lambda i: (0, i)),
            ],
            out_specs=[
                pl.BlockSpec((GW, D), lambda i: (i, 0)),
                pl.BlockSpec((GW, D), lambda i: (i, 0)),
            ],
            core_axis_name=("core", "subcore"),
            dimension_semantics=(pltpu.PARALLEL,),
        )(s_hbm, r_hbm, gs_hbm, gr_hbm)

    return gk(ps, pr, senders.reshape(1, E), receivers.reshape(1, E))


def _edge_body(gs_ref, gr_ref, ef_ref, w0e_ref, b0_ref, w1_ref, b1_ref,
               g_ref, beta_ref, y_ref, out_ref):
    ef = ef_ref[...]
    x = gs_ref[...] + gr_ref[...] + _dot(ef, w0e_ref[...]) + b0_ref[...]
    h = jnp.maximum(x, 0.0)
    y = _dot(h, w1_ref[...]) + b1_ref[...]
    mu = jnp.mean(y, axis=1, keepdims=True)
    d = y - mu
    var = jnp.mean(d * d, axis=1, keepdims=True)
    yln = d * lax.rsqrt(var + 1e-5) * g_ref[...] + beta_ref[...]
    y_ref[...] = yln
    out_ref[...] = yln + ef


def _edge_mlp(gs, gr, ef, w0e, b0, w1, b1, g, beta, be=2000):
    grid = (E // be,)
    row = lambda i: (i, 0)
    full = lambda i: (0, 0)
    return pl.pallas_call(
        _edge_body,
        grid=grid,
        in_specs=[
            pl.BlockSpec((be, D), row),
            pl.BlockSpec((be, D), row),
            pl.BlockSpec((be, D), row),
            pl.BlockSpec((D, D), full),
            pl.BlockSpec((1, D), full),
            pl.BlockSpec((D, D), full),
            pl.BlockSpec((1, D), full),
            pl.BlockSpec((1, D), full),
            pl.BlockSpec((1, D), full),
        ],
        out_specs=[
            pl.BlockSpec((be, D), row),
            pl.BlockSpec((be, D), row),
        ],
        out_shape=[
            jax.ShapeDtypeStruct((E, D), jnp.float32),
            jax.ShapeDtypeStruct((E, D), jnp.float32),
        ],
    )(gs, gr, ef, w0e, b0, w1, b1, g, beta)


def _sc_scatter(y, receivers):
    mesh = plsc.VectorSubcoreMesh(core_axis_name="core",
                                  subcore_axis_name="subcore")

    @functools.partial(
        pl.kernel,
        out_type=jax.ShapeDtypeStruct((NC, N, D), jnp.float32),
        mesh=mesh,
        scratch_types=[
            pltpu.VMEM((SCH,), jnp.int32),
            pltpu.VMEM((SCH, D), jnp.float32),
            pltpu.VMEM((ZR, D), jnp.float32),
            pltpu.VMEM_SHARED((N, D), jnp.float32),
        ],
    )
    def sk(y_hbm, r_hbm, out_hbm, idx_v, rows_v, zbuf_v, acc_sh):
        cid = lax.axis_index("core")
        sid = lax.axis_index("subcore")

        @pl.loop(0, ZR)
        def _(r):
            @pl.loop(0, D // 16)
            def _(c):
                zbuf_v[r, pl.ds(c * 16, 16)] = jnp.zeros((16,), jnp.float32)

        @pl.loop(0, RPT // ZR)
        def _(j):
            pltpu.sync_copy(zbuf_v, acc_sh.at[pl.ds(sid * RPT + j * ZR, ZR)])

        plsc.subcore_barrier()

        wid = cid * NS + sid

        @pl.loop(0, SNCH)
        def _(c):
            base = wid * EPW + c * SCH
            pltpu.sync_copy(r_hbm.at[pl.ds(base, SCH)], idx_v)
            pltpu.sync_copy(y_hbm.at[pl.ds(base, SCH)], rows_v)
            pltpu.sync_copy(rows_v, acc_sh.at[idx_v], add=True)

        plsc.subcore_barrier()

        @pl.loop(0, RPT // ZR)
        def _(j):
            r0 = sid * RPT + j * ZR
            pltpu.sync_copy(acc_sh.at[pl.ds(r0, ZR)], zbuf_v)
            pltpu.sync_copy(zbuf_v, out_hbm.at[cid, pl.ds(r0, ZR)])

    return sk(y, receivers)


def _node_body(nf_ref, a0_ref, a1_ref, w0a_ref, w0b_ref, b0_ref, w1_ref,
               b1_ref, g_ref, beta_ref, out_ref):
    nf = nf_ref[...]
    acc = a0_ref[...] + a1_ref[...]
    x = _dot(nf, w0a_ref[...]) + _dot(acc, w0b_ref[...]) + b0_ref[...]
    h = jnp.maximum(x, 0.0)
    y = _dot(h, w1_ref[...]) + b1_ref[...]
    mu = jnp.mean(y, axis=1, keepdims=True)
    d = y - mu
    var = jnp.mean(d * d, axis=1, keepdims=True)
    out_ref[...] = d * lax.rsqrt(var + 1e-5) * g_ref[...] + beta_ref[...] + nf


def _node_mlp(nf, a0, a1, w0a, w0b, b0, w1, b1, g, beta, bn=2000):
    grid = (N // bn,)
    row = lambda i: (i, 0)
    full = lambda i: (0, 0)
    return pl.pallas_call(
        _node_body,
        grid=grid,
        in_specs=[
            pl.BlockSpec((bn, D), row),
            pl.BlockSpec((bn, D), row),
            pl.BlockSpec((bn, D), row),
            pl.BlockSpec((D, D), full),
            pl.BlockSpec((D, D), full),
            pl.BlockSpec((1, D), full),
            pl.BlockSpec((D, D), full),
            pl.BlockSpec((1, D), full),
            pl.BlockSpec((1, D), full),
            pl.BlockSpec((1, D), full),
        ],
        out_specs=pl.BlockSpec((bn, D), row),
        out_shape=jax.ShapeDtypeStruct((N, D), jnp.float32),
    )(nf, a0, a1, w0a, w0b, b0, w1, b1, g, beta)


def kernel(senders, receivers, node_features, edge_features, params):
    nf = node_features.reshape(N, D)
    ef = edge_features.reshape(E, D)
    s = senders.reshape(E).astype(jnp.int32)
    r = receivers.reshape(E).astype(jnp.int32)

    pe = params["edge"]
    pn = params["node"]
    w0 = pe["W0"]
    w0s, w0r, w0e = w0[:D], w0[D:2 * D], w0[2 * D:]
    row = lambda v: v.reshape(1, D)

    ps, pr = _project(nf, w0s, w0r)
    gs, gr = _sc_gather(ps, pr, s, r)
    y, new_edge = _edge_mlp(gs, gr, ef, w0e, row(pe["b0"]), pe["W1"],
                            row(pe["b1"]), row(pe["g"]), row(pe["beta"]))
    partials = _sc_scatter(y, r)
    w0n = pn["W0"]
    new_node = _node_mlp(nf, partials[0], partials[1], w0n[:D], w0n[D:],
                         row(pn["b0"]), pn["W1"], row(pn["b1"]),
                         row(pn["g"]), row(pn["beta"]))

    return (new_node.reshape(1, N, D), new_edge.reshape(1, E, D))


# speedup vs baseline: 2.7992x; 2.7992x over previous
import functools

import jax
import jax.numpy as jnp
from jax import lax
from jax.experimental import pallas as pl
from jax.experimental.pallas import tpu as pltpu
from jax.experimental.pallas import tpu_sc as plsc

N = 10000
E = 320000
D = 128

NC = 2
NS = 16
NW = NC * NS
GW = 128
EPW = E // NW
SCH = 80
SNCH = EPW // SCH
NP = 10240
RPT = NP // NS
ZR = 128

_PREC = lax.Precision.HIGHEST


def _dot(a, b):
    return lax.dot_general(a, b, (((1,), (0,)), ((), ())), precision=_PREC,
                           preferred_element_type=jnp.float32)


def _project_body(nf_ref, w0s_ref, w0r_ref, ps_ref, pr_ref):
    nf = nf_ref[...]
    ps_ref[...] = _dot(nf, w0s_ref[...])
    pr_ref[...] = _dot(nf, w0r_ref[...])


def _project(nf, w0s, w0r, bn=2000):
    grid = (N // bn,)
    return pl.pallas_call(
        _project_body,
        grid=grid,
        in_specs=[
            pl.BlockSpec((bn, D), lambda i: (i, 0)),
            pl.BlockSpec((D, D), lambda i: (0, 0)),
            pl.BlockSpec((D, D), lambda i: (0, 0)),
        ],
        out_specs=[
            pl.BlockSpec((bn, D), lambda i: (i, 0)),
            pl.BlockSpec((bn, D), lambda i: (i, 0)),
        ],
        out_shape=[
            jax.ShapeDtypeStruct((N, D), jnp.float32),
            jax.ShapeDtypeStruct((N, D), jnp.float32),
        ],
    )(nf, w0s, w0r)


def _sc_gather(ps, pr, senders, receivers):
    mesh = plsc.VectorSubcoreMesh(core_axis_name="core",
                                  subcore_axis_name="subcore")

    @functools.partial(
        pl.kernel,
        out_type=(
            jax.ShapeDtypeStruct((E, D), jnp.float32),
            jax.ShapeDtypeStruct((E, D), jnp.float32),
        ),
        mesh=mesh,
    )
    def gk(ps_hbm, pr_hbm, s_hbm, r_hbm, gs_hbm, gr_hbm):
        def body(si_v, ri_v, gs_v, gr_v):
            pltpu.sync_copy(ps_hbm.at[si_v.at[0]], gs_v)
            pltpu.sync_copy(pr_hbm.at[ri_v.at[0]], gr_v)

        pltpu.emit_pipeline(
            body,
            grid=(E // GW,),
            in_specs=[
                pl.BlockSpec((1, GW), lambda i: (0, i)),
                pl.BlockSpec((1, GW), lambda i: (0, i)),
            ],
            out_specs=[
                pl.BlockSpec((GW, D), lambda i: (i, 0)),
                pl.BlockSpec((GW, D), lambda i: (i, 0)),
            ],
            core_axis_name=("core", "subcore"),
            dimension_semantics=(pltpu.PARALLEL,),
        )(s_hbm, r_hbm, gs_hbm, gr_hbm)

    return gk(ps, pr, senders.reshape(1, E), receivers.reshape(1, E))


def _edge_body(gs_ref, gr_ref, ef_ref, w0e_ref, b0_ref, w1_ref, b1_ref,
               g_ref, beta_ref, y_ref, out_ref):
    ef = ef_ref[...]
    x = gs_ref[...] + gr_ref[...] + _dot(ef, w0e_ref[...]) + b0_ref[...]
    h = jnp.maximum(x, 0.0)
    y = _dot(h, w1_ref[...]) + b1_ref[...]
    mu = jnp.mean(y, axis=1, keepdims=True)
    d = y - mu
    var = jnp.mean(d * d, axis=1, keepdims=True)
    yln = d * lax.rsqrt(var + 1e-5) * g_ref[...] + beta_ref[...]
    y_ref[...] = yln
    out_ref[...] = yln + ef


def _edge_mlp(gs, gr, ef, w0e, b0, w1, b1, g, beta, be=2000):
    grid = (E // be,)
    row = lambda i: (i, 0)
    full = lambda i: (0, 0)
    return pl.pallas_call(
        _edge_body,
        grid=grid,
        in_specs=[
            pl.BlockSpec((be, D), row),
            pl.BlockSpec((be, D), row),
            pl.BlockSpec((be, D), row),
            pl.BlockSpec((D, D), full),
            pl.BlockSpec((1, D), full),
            pl.BlockSpec((D, D), full),
            pl.BlockSpec((1, D), full),
            pl.BlockSpec((1, D), full),
            pl.BlockSpec((1, D), full),
        ],
        out_specs=[
            pl.BlockSpec((be, D), row),
            pl.BlockSpec((be, D), row),
        ],
        out_shape=[
            jax.ShapeDtypeStruct((E, D), jnp.float32),
            jax.ShapeDtypeStruct((E, D), jnp.float32),
        ],
    )(gs, gr, ef, w0e, b0, w1, b1, g, beta)


def _sc_scatter(y, receivers):
    mesh = plsc.VectorSubcoreMesh(core_axis_name="core",
                                  subcore_axis_name="subcore")

    @functools.partial(
        pl.kernel,
        out_type=jax.ShapeDtypeStruct((NC, NP, D), jnp.float32),
        mesh=mesh,
        scratch_types=[
            pltpu.VMEM((SCH,), jnp.int32),
            pltpu.VMEM((SCH, D), jnp.float32),
            pltpu.VMEM((ZR, D), jnp.float32),
            pltpu.VMEM_SHARED((NP, D), jnp.float32),
        ],
    )
    def sk(y_hbm, r_hbm, out_hbm, idx_v, rows_v, zbuf_v, acc_sh):
        cid = lax.axis_index("core")
        sid = lax.axis_index("subcore")

        @pl.loop(0, ZR)
        def _(r):
            @pl.loop(0, D // 16)
            def _(c):
                zbuf_v[r, pl.ds(c * 16, 16)] = jnp.zeros((16,), jnp.float32)

        @pl.loop(0, RPT // ZR)
        def _(j):
            pltpu.sync_copy(zbuf_v, acc_sh.at[pl.ds(sid * RPT + j * ZR, ZR)])

        plsc.subcore_barrier()

        wid = cid * NS + sid

        @pl.loop(0, SNCH)
        def _(c):
            base = wid * EPW + c * SCH
            pltpu.sync_copy(r_hbm.at[pl.ds(base, SCH)], idx_v)
            pltpu.sync_copy(y_hbm.at[pl.ds(base, SCH)], rows_v)
            pltpu.sync_copy(rows_v, acc_sh.at[idx_v], add=True)

        plsc.subcore_barrier()

        @pl.loop(0, RPT // ZR)
        def _(j):
            r0 = sid * RPT + j * ZR
            pltpu.sync_copy(acc_sh.at[pl.ds(r0, ZR)], zbuf_v)
            pltpu.sync_copy(zbuf_v, out_hbm.at[cid, pl.ds(r0, ZR)])

    return sk(y, receivers)


def _node_body(nf_ref, a0_ref, a1_ref, w0a_ref, w0b_ref, b0_ref, w1_ref,
               b1_ref, g_ref, beta_ref, out_ref):
    nf = nf_ref[...]
    acc = a0_ref[...] + a1_ref[...]
    x = _dot(nf, w0a_ref[...]) + _dot(acc, w0b_ref[...]) + b0_ref[...]
    h = jnp.maximum(x, 0.0)
    y = _dot(h, w1_ref[...]) + b1_ref[...]
    mu = jnp.mean(y, axis=1, keepdims=True)
    d = y - mu
    var = jnp.mean(d * d, axis=1, keepdims=True)
    out_ref[...] = d * lax.rsqrt(var + 1e-5) * g_ref[...] + beta_ref[...] + nf


def _node_mlp(nf, a0, a1, w0a, w0b, b0, w1, b1, g, beta, bn=2000):
    grid = (N // bn,)
    row = lambda i: (i, 0)
    full = lambda i: (0, 0)
    return pl.pallas_call(
        _node_body,
        grid=grid,
        in_specs=[
            pl.BlockSpec((bn, D), row),
            pl.BlockSpec((bn, D), row),
            pl.BlockSpec((bn, D), row),
            pl.BlockSpec((D, D), full),
            pl.BlockSpec((D, D), full),
            pl.BlockSpec((1, D), full),
            pl.BlockSpec((D, D), full),
            pl.BlockSpec((1, D), full),
            pl.BlockSpec((1, D), full),
            pl.BlockSpec((1, D), full),
        ],
        out_specs=pl.BlockSpec((bn, D), row),
        out_shape=jax.ShapeDtypeStruct((N, D), jnp.float32),
    )(nf, a0, a1, w0a, w0b, b0, w1, b1, g, beta)


def kernel(senders, receivers, node_features, edge_features, params):
    nf = node_features.reshape(N, D)
    ef = edge_features.reshape(E, D)
    s = senders.reshape(E).astype(jnp.int32)
    r = receivers.reshape(E).astype(jnp.int32)

    pe = params["edge"]
    pn = params["node"]
    w0 = pe["W0"]
    w0s, w0r, w0e = w0[:D], w0[D:2 * D], w0[2 * D:]
    row = lambda v: v.reshape(1, D)

    ps, pr = _project(nf, w0s, w0r)
    gs, gr = _sc_gather(ps, pr, s, r)
    y, new_edge = _edge_mlp(gs, gr, ef, w0e, row(pe["b0"]), pe["W1"],
                            row(pe["b1"]), row(pe["g"]), row(pe["beta"]))
    partials = _sc_scatter(y, r)
    w0n = pn["W0"]
    new_node = _node_mlp(nf, partials[0, :N], partials[1, :N], w0n[:D], w0n[D:],
                         row(pn["b0"]), pn["W1"], row(pn["b1"]),
                         row(pn["g"]), row(pn["beta"]))

    return (new_node.reshape(1, N, D), new_edge.reshape(1, E, D))


# speedup vs baseline: 3.7366x; 1.3349x over previous
import functools

import jax
import jax.numpy as jnp
from jax import lax
from jax.experimental import pallas as pl
from jax.experimental.pallas import tpu as pltpu
from jax.experimental.pallas import tpu_sc as plsc

N = 10000
E = 320000
D = 128

NC = 2
NS = 16
NW = NC * NS
GW = 128
EPW = E // NW
SCH = 80
SNCH = EPW // SCH
NP = 10240
RPT = NP // NS
ZR = 128

_PREC = lax.Precision.DEFAULT


def _dot(a, b):
    return lax.dot_general(a, b, (((1,), (0,)), ((), ())), precision=_PREC,
                           preferred_element_type=jnp.float32)


def _project_body(nf_ref, w0s_ref, w0r_ref, ps_ref, pr_ref):
    nf = nf_ref[...]
    ps_ref[...] = _dot(nf, w0s_ref[...])
    pr_ref[...] = _dot(nf, w0r_ref[...])


def _project(nf, w0s, w0r, bn=2000):
    grid = (N // bn,)
    return pl.pallas_call(
        _project_body,
        grid=grid,
        in_specs=[
            pl.BlockSpec((bn, D), lambda i: (i, 0)),
            pl.BlockSpec((D, D), lambda i: (0, 0)),
            pl.BlockSpec((D, D), lambda i: (0, 0)),
        ],
        out_specs=[
            pl.BlockSpec((bn, D), lambda i: (i, 0)),
            pl.BlockSpec((bn, D), lambda i: (i, 0)),
        ],
        out_shape=[
            jax.ShapeDtypeStruct((N, D), jnp.float32),
            jax.ShapeDtypeStruct((N, D), jnp.float32),
        ],
    )(nf, w0s, w0r)


def _sc_gather(ps, pr, senders, receivers):
    mesh = plsc.VectorSubcoreMesh(core_axis_name="core",
                                  subcore_axis_name="subcore")

    @functools.partial(
        pl.kernel,
        out_type=(
            jax.ShapeDtypeStruct((E, D), jnp.float32),
            jax.ShapeDtypeStruct((E, D), jnp.float32),
        ),
        mesh=mesh,
    )
    def gk(ps_hbm, pr_hbm, s_hbm, r_hbm, gs_hbm, gr_hbm):
        def body(si_v, ri_v, gs_v, gr_v):
            pltpu.sync_copy(ps_hbm.at[si_v.at[0]], gs_v)
            pltpu.sync_copy(pr_hbm.at[ri_v.at[0]], gr_v)

        pltpu.emit_pipeline(
            body,
            grid=(E // GW,),
            in_specs=[
                pl.BlockSpec((1, GW), lambda i: (0, i)),
                pl.BlockSpec((1, GW), lambda i: (0, i)),
            ],
            out_specs=[
                pl.BlockSpec((GW, D), lambda i: (i, 0)),
                pl.BlockSpec((GW, D), lambda i: (i, 0)),
            ],
            core_axis_name=("core", "subcore"),
            dimension_semantics=(pltpu.PARALLEL,),
        )(s_hbm, r_hbm, gs_hbm, gr_hbm)

    return gk(ps, pr, senders.reshape(1, E), receivers.reshape(1, E))


def _edge_body(gs_ref, gr_ref, ef_ref, w0e_ref, b0_ref, w1_ref, b1_ref,
               g_ref, beta_ref, y_ref, out_ref):
    ef = ef_ref[...]
    x = gs_ref[...] + gr_ref[...] + _dot(ef, w0e_ref[...]) + b0_ref[...]
    h = jnp.maximum(x, 0.0)
    y = _dot(h, w1_ref[...]) + b1_ref[...]
    mu = jnp.mean(y, axis=1, keepdims=True)
    d = y - mu
    var = jnp.mean(d * d, axis=1, keepdims=True)
    yln = d * lax.rsqrt(var + 1e-5) * g_ref[...] + beta_ref[...]
    y_ref[...] = yln
    out_ref[...] = yln + ef


def _edge_mlp(gs, gr, ef, w0e, b0, w1, b1, g, beta, be=2000):
    grid = (E // be,)
    row = lambda i: (i, 0)
    full = lambda i: (0, 0)
    return pl.pallas_call(
        _edge_body,
        grid=grid,
        in_specs=[
            pl.BlockSpec((be, D), row),
            pl.BlockSpec((be, D), row),
            pl.BlockSpec((be, D), row),
            pl.BlockSpec((D, D), full),
            pl.BlockSpec((1, D), full),
            pl.BlockSpec((D, D), full),
            pl.BlockSpec((1, D), full),
            pl.BlockSpec((1, D), full),
            pl.BlockSpec((1, D), full),
        ],
        out_specs=[
            pl.BlockSpec((be, D), row),
            pl.BlockSpec((be, D), row),
        ],
        out_shape=[
            jax.ShapeDtypeStruct((E, D), jnp.float32),
            jax.ShapeDtypeStruct((E, D), jnp.float32),
        ],
    )(gs, gr, ef, w0e, b0, w1, b1, g, beta)


def _sc_scatter(y, receivers):
    mesh = plsc.VectorSubcoreMesh(core_axis_name="core",
                                  subcore_axis_name="subcore")

    @functools.partial(
        pl.kernel,
        out_type=jax.ShapeDtypeStruct((NC, NP, D), jnp.float32),
        mesh=mesh,
        scratch_types=[
            pltpu.VMEM((SCH,), jnp.int32),
            pltpu.VMEM((SCH, D), jnp.float32),
            pltpu.VMEM((ZR, D), jnp.float32),
            pltpu.VMEM_SHARED((NP, D), jnp.float32),
        ],
    )
    def sk(y_hbm, r_hbm, out_hbm, idx_v, rows_v, zbuf_v, acc_sh):
        cid = lax.axis_index("core")
        sid = lax.axis_index("subcore")

        @pl.loop(0, ZR)
        def _(r):
            @pl.loop(0, D // 16)
            def _(c):
                zbuf_v[r, pl.ds(c * 16, 16)] = jnp.zeros((16,), jnp.float32)

        @pl.loop(0, RPT // ZR)
        def _(j):
            pltpu.sync_copy(zbuf_v, acc_sh.at[pl.ds(sid * RPT + j * ZR, ZR)])

        plsc.subcore_barrier()

        wid = cid * NS + sid

        @pl.loop(0, SNCH)
        def _(c):
            base = wid * EPW + c * SCH
            pltpu.sync_copy(r_hbm.at[pl.ds(base, SCH)], idx_v)
            pltpu.sync_copy(y_hbm.at[pl.ds(base, SCH)], rows_v)
            pltpu.sync_copy(rows_v, acc_sh.at[idx_v], add=True)

        plsc.subcore_barrier()

        @pl.loop(0, RPT // ZR)
        def _(j):
            r0 = sid * RPT + j * ZR
            pltpu.sync_copy(acc_sh.at[pl.ds(r0, ZR)], zbuf_v)
            pltpu.sync_copy(zbuf_v, out_hbm.at[cid, pl.ds(r0, ZR)])

    return sk(y, receivers)


def _node_body(nf_ref, a0_ref, a1_ref, w0a_ref, w0b_ref, b0_ref, w1_ref,
               b1_ref, g_ref, beta_ref, out_ref):
    nf = nf_ref[...]
    acc = a0_ref[...] + a1_ref[...]
    x = _dot(nf, w0a_ref[...]) + _dot(acc, w0b_ref[...]) + b0_ref[...]
    h = jnp.maximum(x, 0.0)
    y = _dot(h, w1_ref[...]) + b1_ref[...]
    mu = jnp.mean(y, axis=1, keepdims=True)
    d = y - mu
    var = jnp.mean(d * d, axis=1, keepdims=True)
    out_ref[...] = d * lax.rsqrt(var + 1e-5) * g_ref[...] + beta_ref[...] + nf


def _node_mlp(nf, a0, a1, w0a, w0b, b0, w1, b1, g, beta, bn=2000):
    grid = (N // bn,)
    row = lambda i: (i, 0)
    full = lambda i: (0, 0)
    return pl.pallas_call(
        _node_body,
        grid=grid,
        in_specs=[
            pl.BlockSpec((bn, D), row),
            pl.BlockSpec((bn, D), row),
            pl.BlockSpec((bn, D), row),
            pl.BlockSpec((D, D), full),
            pl.BlockSpec((D, D), full),
            pl.BlockSpec((1, D), full),
            pl.BlockSpec((D, D), full),
            pl.BlockSpec((1, D), full),
            pl.BlockSpec((1, D), full),
            pl.BlockSpec((1, D), full),
        ],
        out_specs=pl.BlockSpec((bn, D), row),
        out_shape=jax.ShapeDtypeStruct((N, D), jnp.float32),
    )(nf, a0, a1, w0a, w0b, b0, w1, b1, g, beta)


def kernel(senders, receivers, node_features, edge_features, params):
    nf = node_features.reshape(N, D)
    ef = edge_features.reshape(E, D)
    s = senders.reshape(E).astype(jnp.int32)
    r = receivers.reshape(E).astype(jnp.int32)

    pe = params["edge"]
    pn = params["node"]
    w0 = pe["W0"]
    w0s, w0r, w0e = w0[:D], w0[D:2 * D], w0[2 * D:]
    row = lambda v: v.reshape(1, D)

    ps, pr = _project(nf, w0s, w0r)
    gs, gr = _sc_gather(ps, pr, s, r)
    y, new_edge = _edge_mlp(gs, gr, ef, w0e, row(pe["b0"]), pe["W1"],
                            row(pe["b1"]), row(pe["g"]), row(pe["beta"]))
    partials = _sc_scatter(y, r)
    w0n = pn["W0"]
    new_node = _node_mlp(nf, partials[0, :N], partials[1, :N], w0n[:D], w0n[D:],
                         row(pn["b0"]), pn["W1"], row(pn["b1"]),
                         row(pn["g"]), row(pn["beta"]))

    return (new_node.reshape(1, N, D), new_edge.reshape(1, E, D))
